# baseline (speedup 1.0000x reference)
import jax
import jax.numpy as jnp
from jax import lax
from jax.experimental import pallas as pl
from jax.experimental.pallas import tpu as pltpu

N = 512
D = 32
BPG = 8
NB = N // BPG
NBUF = 10
I_BLK = 64


def _fused(idx, lower_t, higher_t, p_lo, p_hi):

    def body(idx_ref, lo_hbm, hi_hbm, pl_ref, ph_ref, out_ref,
             st_lo, st_hi, glo, ghi, sem_lo, sem_hi):
        lane_w = lax.rem(
            lax.broadcasted_iota(jnp.int32, (1, BPG * 128), 1), 128)

        def issue(b):
            buf = b % NBUF
            cps = []
            for k in range(BPG):
                v = idx_ref[b * BPG + k]
                base = pl.multiple_of((v // 128) * 128, 128)
                cps.append(pltpu.make_async_copy(
                    lo_hbm.at[:, pl.ds(base, 128)],
                    st_lo.at[buf, :, pl.ds(128 * k, 128)], sem_lo))
                cps.append(pltpu.make_async_copy(
                    hi_hbm.at[:, pl.ds(base, 128)],
                    st_hi.at[buf, :, pl.ds(128 * k, 128)], sem_hi))
            for cp in cps:
                cp.start()
            return cps

        pending = {i: issue(i) for i in range(NBUF - 1)}
        for b in range(NB):
            nxt = b + NBUF - 1
            if nxt < NB:
                pending[nxt] = issue(nxt)
            for cp in pending.pop(b):
                cp.wait()
            buf = b % NBUF
            cvec = jnp.concatenate(
                [jnp.full((1, 128), lax.rem(idx_ref[b * BPG + k], 128),
                          jnp.int32) for k in range(BPG)], axis=1)
            m = lane_w == cvec
            ext_lo = jnp.sum(
                jnp.reshape(jnp.where(m, st_lo[buf], 0.0), (D, BPG, 128)),
                axis=2)
            ext_hi = jnp.sum(
                jnp.reshape(jnp.where(m, st_hi[buf], 0.0), (D, BPG, 128)),
                axis=2)
            glo[b, :, :] = ext_lo
            ghi[b, :, :] = ext_hi

        lo_tj = jnp.concatenate([glo[b] for b in range(NB)], axis=1)
        hi_tj = jnp.concatenate([ghi[b] for b in range(NB)], axis=1)
        lo_all = jnp.transpose(lo_tj)
        hi_all = jnp.transpose(hi_tj)
        p_l = pl_ref[...]
        p_h = ph_ref[...]
        zero = jnp.float32(0.0)
        total = (jnp.sum(jnp.maximum(p_l - lo_all, zero))
                 + jnp.sum(jnp.maximum(hi_all - p_h, zero))
                 + jnp.sum(jnp.maximum(p_l - hi_all, zero))
                 + jnp.sum(jnp.maximum(lo_all - p_h, zero))
                 - jnp.sum(jnp.maximum(hi_all - lo_all, zero)))
        lo_j = lo_tj[None, :, :]
        hi_j = hi_tj[None, :, :]
        for c in range(N // I_BLK):
            lo_b = lo_all[c * I_BLK:(c + 1) * I_BLK, :, None]
            hi_b = hi_all[c * I_BLK:(c + 1) * I_BLK, :, None]
            ov = jnp.minimum(hi_b, hi_j) - jnp.maximum(lo_b, lo_j)
            total += jnp.sum(jnp.maximum(ov, zero))
        out_ref[0, 0] = total

    return pl.pallas_call(
        body,
        grid_spec=pltpu.PrefetchScalarGridSpec(
            num_scalar_prefetch=1,
            grid=(1,),
            in_specs=[
                pl.BlockSpec(memory_space=pl.ANY),
                pl.BlockSpec(memory_space=pl.ANY),
                pl.BlockSpec((1, D), lambda s, i: (0, 0)),
                pl.BlockSpec((1, D), lambda s, i: (0, 0)),
            ],
            out_specs=pl.BlockSpec(memory_space=pltpu.SMEM),
            scratch_shapes=[
                pltpu.VMEM((NBUF, D, BPG * 128), jnp.float32),
                pltpu.VMEM((NBUF, D, BPG * 128), jnp.float32),
                pltpu.VMEM((NB, D, BPG), jnp.float32),
                pltpu.VMEM((NB, D, BPG), jnp.float32),
                pltpu.SemaphoreType.DMA,
                pltpu.SemaphoreType.DMA,
            ],
        ),
        out_shape=jax.ShapeDtypeStruct((1, 1), jnp.float32),
    )(idx, lower_t, higher_t, p_lo, p_hi)


def kernel(idIndexes, omegaEmb, epoch, childrenLowerEmbedding,
           childrenHigherEmbedding, parentsEmbL_, parentsEmbH_,
           parentRange, leavesRatio):
    idx = idIndexes.astype(jnp.int32)
    loss = _fused(
        idx, childrenLowerEmbedding.T, childrenHigherEmbedding.T,
        parentsEmbL_.reshape(1, D), parentsEmbH_.reshape(1, D),
    )
    return jnp.reshape(loss, ())
